# baseline (speedup 1.0000x reference)
import functools

import jax
import jax.numpy as jnp
from jax import lax
from jax.experimental import pallas as pl
from jax.experimental.pallas import tpu as pltpu
from jax.experimental.pallas import tpu_sc as plsc

B = 16384
L = 50
D = 32
NW = 32
EPW = B // NW
CB = 32
NCH = EPW // CB
ROWS = CB * L
HALF = D // 2


def kernel(batch, table):
    mesh = plsc.VectorSubcoreMesh(core_axis_name="c", subcore_axis_name="s")

    @functools.partial(
        pl.kernel,
        mesh=mesh,
        out_type=jax.ShapeDtypeStruct((B * D,), jnp.float32),
        scratch_types=[
            pltpu.VMEM((CB, L), jnp.int32),
            pltpu.VMEM((ROWS, D), jnp.float32),
            pltpu.VMEM((CB * D,), jnp.float32),
            pltpu.SemaphoreType.DMA,
        ],
        compiler_params=pltpu.CompilerParams(use_tc_tiling_on_sc=False),
    )
    def bow(idx_hbm, table_hbm, out_hbm, idx_v, rows_v, out_v, gsem):
        wid = lax.axis_index("s") * 2 + lax.axis_index("c")

        def chunk_body(c, carry):
            b0 = wid * EPW + c * CB
            pltpu.sync_copy(idx_hbm.at[pl.ds(b0, CB)], idx_v)
            gcopies = [
                pltpu.async_copy(
                    table_hbm.at[idx_v.at[j]],
                    rows_v.at[pl.ds(j * L, L)],
                    gsem,
                )
                for j in range(CB)
            ]
            for cp in gcopies:
                cp.wait()

            def elem_body(e, carry2):
                r0 = e * L
                a0 = rows_v[r0, 0:HALF] + rows_v[r0 + 1, 0:HALF]
                b0_ = rows_v[r0, HALF:D] + rows_v[r0 + 1, HALF:D]
                a1 = rows_v[r0 + 2, 0:HALF] + rows_v[r0 + 3, 0:HALF]
                b1 = rows_v[r0 + 2, HALF:D] + rows_v[r0 + 3, HALF:D]
                for l in range(4, L, 2):
                    a0 = a0 + rows_v[r0 + l, 0:HALF]
                    b0_ = b0_ + rows_v[r0 + l, HALF:D]
                    a1 = a1 + rows_v[r0 + l + 1, 0:HALF]
                    b1 = b1 + rows_v[r0 + l + 1, HALF:D]
                o0 = e * D
                out_v[pl.ds(o0, HALF)] = (a0 + a1) * (1.0 / L)
                out_v[pl.ds(o0 + HALF, HALF)] = (b0_ + b1) * (1.0 / L)
                return carry2

            lax.fori_loop(0, CB, elem_body, 0)
            pltpu.sync_copy(out_v, out_hbm.at[pl.ds(b0 * D, CB * D)])
            return carry

        lax.fori_loop(0, NCH, chunk_body, 0)

    return bow(batch, table).reshape(B, D)
